# baseline (speedup 1.0000x reference)
import functools

import numpy as np
import jax
import jax.numpy as jnp
from jax import lax
from jax.experimental import pallas as pl
from jax.experimental.pallas import tpu as pltpu
from jax.experimental.pallas import tpu_sc as plsc

_A = np.float32(0.25)
_VOCAB = 1_000_000
_BATCH = 16384
_HIST = 200
_N = _BATCH * _HIST
_NC = 2
_NS = 16
_NW = _NC * _NS
_COLS_W = _BATCH // _NW
_CH_R = 8
_CH = _CH_R * _COLS_W
_NCHUNK = _HIST // _CH_R
_NSLOT = 4
_UNROLL = 8
_STAGERS = 10
_TSLICE = _VOCAB // _STAGERS
_TCHUNK = 4000

_thr_cache = None


def _rotl32(x, r):
    return ((x << np.uint32(r)) | (x >> np.uint32(32 - r))).astype(np.uint32)


def _threefry2x32(k0, k1, x0, x1):
    x0 = np.asarray(x0, np.uint32).copy()
    x1 = np.asarray(x1, np.uint32).copy()
    ks = [np.uint32(k0), np.uint32(k1),
          np.uint32(np.uint32(k0) ^ np.uint32(k1) ^ np.uint32(0x1BD11BDA))]
    rot = [(13, 15, 26, 6), (17, 29, 16, 24)]
    x0 = (x0 + ks[0]).astype(np.uint32)
    x1 = (x1 + ks[1]).astype(np.uint32)
    for i in range(5):
        for r in rot[i % 2]:
            x0 = (x0 + x1).astype(np.uint32)
            x1 = _rotl32(x1, r)
            x1 = (x1 ^ x0).astype(np.uint32)
        x0 = (x0 + ks[(i + 1) % 3]).astype(np.uint32)
        x1 = (x1 + ks[(i + 2) % 3] + np.uint32(i + 1)).astype(np.uint32)
    return x0, x1


def _uniform_draw():
    f0, f1 = _threefry2x32(0, 0, np.zeros(1, np.uint32),
                           np.full(1, 1234, np.uint32))
    i = np.arange(_N, dtype=np.uint32)
    b1, b2 = _threefry2x32(f0[0], f1[0], np.zeros(_N, np.uint32), i)
    bits = (b1 ^ b2).astype(np.uint32)
    return ((bits >> np.uint32(9)) | np.uint32(0x3F800000)).view(np.float32) \
        - np.float32(1.0)


def _drop_thresholds():
    global _thr_cache
    if _thr_cache is None:
        u = _uniform_draw()
        lo = np.zeros(u.shape, np.uint32)
        hi = np.full(u.shape, 0x7F800000, np.uint32)
        for _ in range(31):
            mid = (lo + hi) // np.uint32(2)
            c = mid.view(np.float32)
            pred = (_A / (_A + c)) <= u
            hi = np.where(pred, mid, hi)
            lo = np.where(pred, lo, mid + np.uint32(1))
        _thr_cache = np.ascontiguousarray(
            hi.view(np.float32).reshape(_BATCH, _HIST).T)
    return _thr_cache


def _sc_body(idx_hbm, thr_hbm, tbl_hbm, tr_hbm, out_hbm,
             tbl_sh, stg_0, stg_1, stg_2, stg_3,
             idx1_0, idx1_1, idx1_2, idx1_3,
             thr_0, thr_1, thr_2, thr_3,
             cnt_0, cnt_1, cnt_2, cnt_3, tr_v,
             sin0, sin1, sin2, sin3, sg0, sg1, sg2, sg3,
             sout0, sout1, sout2, sout3):
    cid = lax.axis_index("c")
    sid = lax.axis_index("s")
    wid = sid * _NC + cid
    c0 = wid * _COLS_W
    stg_b = (stg_0, stg_1, stg_2, stg_3)
    idx1_b = (idx1_0, idx1_1, idx1_2, idx1_3)
    thr_b = (thr_0, thr_1, thr_2, thr_3)
    cnt_b = (cnt_0, cnt_1, cnt_2, cnt_3)
    sin_b = (sin0, sin1, sin2, sin3)
    sg_b = (sg0, sg1, sg2, sg3)
    sout_b = (sout0, sout1, sout2, sout3)

    pltpu.sync_copy(tr_hbm, tr_v)
    tr16 = tr_v[...]

    def start_in(k):
        b = k % _NSLOT
        rows = pl.ds(k * _CH_R, _CH_R)
        cols = pl.ds(c0, _COLS_W)
        return (pltpu.async_copy(idx_hbm.at[rows, cols], stg_b[b], sin_b[b]),
                pltpu.async_copy(thr_hbm.at[rows, cols], thr_b[b], sin_b[b]))

    def repack(k):
        b = k % _NSLOT
        idx2, idx1 = stg_b[b], idx1_b[b]

        def body(v, carry):
            for u in range(_UNROLL):
                j = v * _UNROLL + u
                r = j // (_COLS_W // 16)
                c = (j % (_COLS_W // 16)) * 16
                idx1[pl.ds(r * _COLS_W + c, 16)] = idx2[r, pl.ds(c, 16)]
            return carry

        lax.fori_loop(0, _CH // 16 // _UNROLL, body, 0)

    def start_gather(k):
        b = k % _NSLOT
        src = tbl_hbm if k < 2 else tbl_sh
        return pltpu.async_copy(src.at[idx1_b[b]], cnt_b[b], sg_b[b])

    def start_out(k):
        b = k % _NSLOT
        rows = pl.ds(k * _CH_R, _CH_R)
        cols = pl.ds(c0, _COLS_W)
        return pltpu.async_copy(stg_b[b], out_hbm.at[rows, cols], sout_b[b])

    def compute(k):
        b = k % _NSLOT
        cnt_v, thr_v, idx_v, out_v = cnt_b[b], thr_b[b], idx1_b[b], stg_b[b]
        zeros = jnp.zeros((16,), jnp.int32)

        def body(v, carry):
            for u in range(_UNROLL):
                j = v * _UNROLL + u
                r = j // (_COLS_W // 16)
                c = (j % (_COLS_W // 16)) * 16
                sl1 = pl.ds(r * _COLS_W + c, 16)
                sl2 = (r, pl.ds(c, 16))
                drop = cnt_v[sl1] < thr_v[sl2] * tr16
                out_v[sl2] = jnp.where(drop, zeros, idx_v[sl1])
            return carry

        lax.fori_loop(0, _CH // 16 // _UNROLL, body, 0)

    ins = {k: start_in(k) for k in range(3)}

    @pl.when(sid < _STAGERS)
    def _():
        sbase = sid * _TSLICE
        nsub = _TSLICE // _TCHUNK
        bnc = (cnt_2, cnt_3)
        sem = (sg2, sg3)

        def rd(j):
            o = sbase + j * _TCHUNK
            return pltpu.async_copy(tbl_hbm.at[pl.ds(o, _TCHUNK)],
                                    bnc[j % 2].at[pl.ds(0, _TCHUNK)],
                                    sem[j % 2])

        def wr(j):
            o = sbase + j * _TCHUNK
            return pltpu.async_copy(bnc[j % 2].at[pl.ds(0, _TCHUNK)],
                                    tbl_sh.at[pl.ds(o, _TCHUNK)],
                                    sem[j % 2])

        rds = {0: rd(0)}
        wrs = {}
        for j in range(nsub):
            rds[j].wait()
            wrs[j] = wr(j)
            if j + 1 < nsub:
                if j - 1 >= 0:
                    wrs[j - 1].wait()
                rds[j + 1] = rd(j + 1)
        wrs[nsub - 2].wait()
        wrs[nsub - 1].wait()

    gathers = {}
    for k in (0, 1):
        for d in ins[k]:
            d.wait()
        repack(k)
        gathers[k] = start_gather(k)
    plsc.subcore_barrier()
    outs = {}
    for k in range(_NCHUNK):
        gathers[k].wait()
        compute(k)
        outs[k] = start_out(k)
        if k + 2 < _NCHUNK:
            for d in ins[k + 2]:
                d.wait()
            repack(k + 2)
            gathers[k + 2] = start_gather(k + 2)
        if k + 3 < _NCHUNK:
            if k - 1 >= 0:
                outs[k - 1].wait()
            ins[k + 3] = start_in(k + 3)
    for k in range(_NCHUNK - 4, _NCHUNK):
        outs[k].wait()


@functools.cache
def _word_dropout_sc():
    return functools.partial(
        pl.kernel,
        out_type=jax.ShapeDtypeStruct((_HIST, _BATCH), jnp.int32),
        mesh=plsc.VectorSubcoreMesh(core_axis_name="c", subcore_axis_name="s"),
        scratch_types=(
            [pltpu.VMEM_SHARED((_VOCAB,), jnp.float32)]
            + [pltpu.VMEM((_CH_R, _COLS_W), jnp.int32)] * 4
            + [pltpu.VMEM((_CH,), jnp.int32)] * 4
            + [pltpu.VMEM((_CH_R, _COLS_W), jnp.float32)] * 4
            + [pltpu.VMEM((_CH,), jnp.float32)] * 4
            + [pltpu.VMEM((16,), jnp.float32)]
            + [pltpu.SemaphoreType.DMA] * 12
        ),
    )(_sc_body)


def kernel(appearance_count, word_idx, train):
    odtype = word_idx.dtype
    idx_t = word_idx.astype(jnp.int32).T
    thr_t = jnp.asarray(_drop_thresholds())
    tr16 = jnp.full((16,), jnp.asarray(train) != 0, jnp.float32)
    out_t = _word_dropout_sc()(idx_t, thr_t, appearance_count, tr16)
    return out_t.T.astype(odtype)


# speedup vs baseline: 1.0504x; 1.0504x over previous
import functools

import numpy as np
import jax
import jax.numpy as jnp
from jax import lax
from jax.experimental import pallas as pl
from jax.experimental.pallas import tpu as pltpu
from jax.experimental.pallas import tpu_sc as plsc

_A = np.float32(0.25)
_VOCAB = 1_000_000
_BATCH = 16384
_HIST = 200
_N = _BATCH * _HIST
_NC = 2
_NS = 16
_NW = _NC * _NS
_COLS_W = _BATCH // _NW
_CH_R = 8
_CH = _CH_R * _COLS_W
_NCHUNK = _HIST // _CH_R
_NSLOT = 4
_UNROLL = 8
_STAGERS = 10
_TSLICE = _VOCAB // _STAGERS
_TCHUNK = 4000

_thr_cache = None


def _rotl32(x, r):
    return ((x << np.uint32(r)) | (x >> np.uint32(32 - r))).astype(np.uint32)


def _threefry2x32(k0, k1, x0, x1):
    x0 = np.asarray(x0, np.uint32).copy()
    x1 = np.asarray(x1, np.uint32).copy()
    ks = [np.uint32(k0), np.uint32(k1),
          np.uint32(np.uint32(k0) ^ np.uint32(k1) ^ np.uint32(0x1BD11BDA))]
    rot = [(13, 15, 26, 6), (17, 29, 16, 24)]
    x0 = (x0 + ks[0]).astype(np.uint32)
    x1 = (x1 + ks[1]).astype(np.uint32)
    for i in range(5):
        for r in rot[i % 2]:
            x0 = (x0 + x1).astype(np.uint32)
            x1 = _rotl32(x1, r)
            x1 = (x1 ^ x0).astype(np.uint32)
        x0 = (x0 + ks[(i + 1) % 3]).astype(np.uint32)
        x1 = (x1 + ks[(i + 2) % 3] + np.uint32(i + 1)).astype(np.uint32)
    return x0, x1


def _uniform_draw():
    f0, f1 = _threefry2x32(0, 0, np.zeros(1, np.uint32),
                           np.full(1, 1234, np.uint32))
    i = np.arange(_N, dtype=np.uint32)
    b1, b2 = _threefry2x32(f0[0], f1[0], np.zeros(_N, np.uint32), i)
    bits = (b1 ^ b2).astype(np.uint32)
    return ((bits >> np.uint32(9)) | np.uint32(0x3F800000)).view(np.float32) \
        - np.float32(1.0)


def _drop_thresholds():
    global _thr_cache
    if _thr_cache is None:
        u = _uniform_draw()
        lo = np.zeros(u.shape, np.uint32)
        hi = np.full(u.shape, 0x7F800000, np.uint32)
        for _ in range(31):
            mid = (lo + hi) // np.uint32(2)
            c = mid.view(np.float32)
            pred = (_A / (_A + c)) <= u
            hi = np.where(pred, mid, hi)
            lo = np.where(pred, lo, mid + np.uint32(1))
        _thr_cache = np.ascontiguousarray(
            hi.view(np.float32).reshape(_BATCH, _HIST).T)
    return _thr_cache


def _sc_body(idx_hbm, thr_hbm, tbl_hbm, tr_hbm, out_hbm,
             tbl_sh, stg_0, stg_1, stg_2, stg_3,
             idx1_0, idx1_1, idx1_2, idx1_3,
             thr_0, thr_1, thr_2, thr_3,
             cnt_0, cnt_1, cnt_2, cnt_3, tr_v,
             sin0, sin1, sin2, sin3, sg0, sg1, sg2, sg3,
             sout0, sout1, sout2, sout3):
    cid = lax.axis_index("c")
    sid = lax.axis_index("s")
    wid = sid * _NC + cid
    c0 = wid * _COLS_W
    stg_b = (stg_0, stg_1, stg_2, stg_3)
    idx1_b = (idx1_0, idx1_1, idx1_2, idx1_3)
    thr_b = (thr_0, thr_1, thr_2, thr_3)
    cnt_b = (cnt_0, cnt_1, cnt_2, cnt_3)
    sin_b = (sin0, sin1, sin2, sin3)
    sg_b = (sg0, sg1, sg2, sg3)
    sout_b = (sout0, sout1, sout2, sout3)

    pltpu.sync_copy(tr_hbm, tr_v)
    tr16 = tr_v[...]

    def start_in(k):
        b = k % _NSLOT
        rows = pl.ds(k * _CH_R, _CH_R)
        cols = pl.ds(c0, _COLS_W)
        return (pltpu.async_copy(idx_hbm.at[rows, cols], stg_b[b], sin_b[b]),
                pltpu.async_copy(thr_hbm.at[rows, cols], thr_b[b], sin_b[b]))

    def repack(k):
        b = k % _NSLOT
        idx2, idx1 = stg_b[b], idx1_b[b]

        def body(v, carry):
            for u in range(_UNROLL):
                j = v * _UNROLL + u
                r = j // (_COLS_W // 16)
                c = (j % (_COLS_W // 16)) * 16
                idx1[pl.ds(r * _COLS_W + c, 16)] = idx2[r, pl.ds(c, 16)]
            return carry

        lax.fori_loop(0, _CH // 16 // _UNROLL, body, 0)

    def start_gather(k):
        b = k % _NSLOT
        return pltpu.async_copy(tbl_sh.at[idx1_b[b]], cnt_b[b], sg_b[b])

    def start_out(k):
        b = k % _NSLOT
        rows = pl.ds(k * _CH_R, _CH_R)
        cols = pl.ds(c0, _COLS_W)
        return pltpu.async_copy(stg_b[b], out_hbm.at[rows, cols], sout_b[b])

    def compute(k):
        b = k % _NSLOT
        cnt_v, thr_v, idx_v, out_v = cnt_b[b], thr_b[b], idx1_b[b], stg_b[b]
        zeros = jnp.zeros((16,), jnp.int32)

        def body(v, carry):
            for u in range(_UNROLL):
                j = v * _UNROLL + u
                r = j // (_COLS_W // 16)
                c = (j % (_COLS_W // 16)) * 16
                sl1 = pl.ds(r * _COLS_W + c, 16)
                sl2 = (r, pl.ds(c, 16))
                drop = cnt_v[sl1] < thr_v[sl2] * tr16
                out_v[sl2] = jnp.where(drop, zeros, idx_v[sl1])
            return carry

        lax.fori_loop(0, _CH // 16 // _UNROLL, body, 0)

    ins = {k: start_in(k) for k in range(3)}

    @pl.when(sid < _STAGERS)
    def _():
        sbase = sid * _TSLICE
        nsub = _TSLICE // _TCHUNK
        bnc = (cnt_2, cnt_3)
        sem = (sg2, sg3)

        def rd(j):
            o = sbase + j * _TCHUNK
            return pltpu.async_copy(tbl_hbm.at[pl.ds(o, _TCHUNK)],
                                    bnc[j % 2].at[pl.ds(0, _TCHUNK)],
                                    sem[j % 2])

        def wr(j):
            o = sbase + j * _TCHUNK
            return pltpu.async_copy(bnc[j % 2].at[pl.ds(0, _TCHUNK)],
                                    tbl_sh.at[pl.ds(o, _TCHUNK)],
                                    sem[j % 2])

        rds = {0: rd(0)}
        wrs = {}
        for j in range(nsub):
            rds[j].wait()
            wrs[j] = wr(j)
            if j + 1 < nsub:
                if j - 1 >= 0:
                    wrs[j - 1].wait()
                rds[j + 1] = rd(j + 1)
        wrs[nsub - 2].wait()
        wrs[nsub - 1].wait()

    plsc.subcore_barrier()
    gathers = {}
    for k in (0, 1):
        for d in ins[k]:
            d.wait()
        repack(k)
        gathers[k] = start_gather(k)
    outs = {}
    for k in range(_NCHUNK):
        gathers[k].wait()
        compute(k)
        outs[k] = start_out(k)
        if k + 2 < _NCHUNK:
            for d in ins[k + 2]:
                d.wait()
            repack(k + 2)
            gathers[k + 2] = start_gather(k + 2)
        if k + 3 < _NCHUNK:
            if k - 1 >= 0:
                outs[k - 1].wait()
            ins[k + 3] = start_in(k + 3)
    for k in range(_NCHUNK - 4, _NCHUNK):
        outs[k].wait()


@functools.cache
def _word_dropout_sc():
    return functools.partial(
        pl.kernel,
        out_type=jax.ShapeDtypeStruct((_HIST, _BATCH), jnp.int32),
        mesh=plsc.VectorSubcoreMesh(core_axis_name="c", subcore_axis_name="s"),
        scratch_types=(
            [pltpu.VMEM_SHARED((_VOCAB,), jnp.float32)]
            + [pltpu.VMEM((_CH_R, _COLS_W), jnp.int32)] * 4
            + [pltpu.VMEM((_CH,), jnp.int32)] * 4
            + [pltpu.VMEM((_CH_R, _COLS_W), jnp.float32)] * 4
            + [pltpu.VMEM((_CH,), jnp.float32)] * 4
            + [pltpu.VMEM((16,), jnp.float32)]
            + [pltpu.SemaphoreType.DMA] * 12
        ),
    )(_sc_body)


def kernel(appearance_count, word_idx, train):
    odtype = word_idx.dtype
    idx_t = word_idx.astype(jnp.int32).T
    thr_t = jnp.asarray(_drop_thresholds())
    tr16 = jnp.full((16,), jnp.asarray(train) != 0, jnp.float32)
    out_t = _word_dropout_sc()(idx_t, thr_t, appearance_count, tr16)
    return out_t.T.astype(odtype)
